# Initial kernel scaffold; baseline (speedup 1.0000x reference)
import jax
import jax.numpy as jnp
from jax.experimental import pallas as pl


def kernel(x, edge_index, edge_attr, params):
    raise NotImplementedError("write your pallas kernel here")


# speedup vs baseline: 3.4169x; 3.4169x over previous
import functools

import jax
import jax.numpy as jnp
from jax import lax
from jax.experimental import pallas as pl
from jax.experimental.pallas import tpu as pltpu
from jax.experimental.pallas import tpu_sc as plsc

N_NODES = 50000
E_EDGES = 800000
D = 64
H = 32
ED = 4
N_LAYERS = 4

ROW_BLK = 512
N_PAD = 50176
N_BLKS = N_PAD // ROW_BLK

IDX_W = 128
CHUNK = 256
SUB = CHUNK // IDX_W
NS = 16
E_PAD = 819200
EPT = E_PAD // NS
NCHUNK = EPT // CHUNK
RPT = N_PAD // NS


def _dot(a, b):
    return jnp.dot(a.astype(jnp.bfloat16), b.astype(jnp.bfloat16),
                   preferred_element_type=jnp.float32)


def _tc_init_body(x_ref, win_ref, bin_ref, wp_ref, bp_ref, wq_ref,
                  h_ref, p0_ref, p1_ref, q0_ref, q1_ref):
    h = _dot(x_ref[...], win_ref[...]) + bin_ref[...]
    h_ref[...] = h
    p = _dot(h, wp_ref[...]) + bp_ref[...]
    q = _dot(h, wq_ref[...])
    p0_ref[...] = p[:, :H]
    p1_ref[...] = p[:, H:]
    q0_ref[...] = q[:, :H]
    q1_ref[...] = q[:, H:]


def _row_spec(w):
    return pl.BlockSpec((ROW_BLK, w), lambda i: (i, 0))


def _full_spec(r, w):
    return pl.BlockSpec((r, w), lambda i: (0, 0))


_tc_init = pl.pallas_call(
    _tc_init_body,
    grid=(N_BLKS,),
    in_specs=[
        _row_spec(11),
        _full_spec(11, D), _full_spec(1, D),
        _full_spec(D, D), _full_spec(1, D), _full_spec(D, D),
    ],
    out_specs=[_row_spec(D), _row_spec(H), _row_spec(H), _row_spec(H), _row_spec(H)],
    out_shape=[
        jax.ShapeDtypeStruct((N_PAD, D), jnp.float32),
        jax.ShapeDtypeStruct((N_PAD, H), jnp.float32),
        jax.ShapeDtypeStruct((N_PAD, H), jnp.float32),
        jax.ShapeDtypeStruct((N_PAD, H), jnp.float32),
        jax.ShapeDtypeStruct((N_PAD, H), jnp.float32),
    ],
)


def _tc_upd_body(h_ref, a0_ref, a1_ref, wu_ref, bu_ref, wp_ref, bp_ref, wq_ref,
                 hn_ref, p0_ref, p1_ref, q0_ref, q1_ref):
    h = h_ref[...]
    u = (_dot(h, wu_ref[0:D, :])
         + _dot(a0_ref[...], wu_ref[D:D + H, :])
         + _dot(a1_ref[...], wu_ref[D + H:2 * D, :])
         + bu_ref[...])
    hn = h + jnp.maximum(u, 0.0)
    hn_ref[...] = hn
    p = _dot(hn, wp_ref[...]) + bp_ref[...]
    q = _dot(hn, wq_ref[...])
    p0_ref[...] = p[:, :H]
    p1_ref[...] = p[:, H:]
    q0_ref[...] = q[:, :H]
    q1_ref[...] = q[:, H:]


_tc_upd = pl.pallas_call(
    _tc_upd_body,
    grid=(N_BLKS,),
    in_specs=[
        _row_spec(D), _row_spec(H), _row_spec(H),
        _full_spec(2 * D, D), _full_spec(1, D),
        _full_spec(D, D), _full_spec(1, D), _full_spec(D, D),
    ],
    out_specs=[_row_spec(D), _row_spec(H), _row_spec(H), _row_spec(H), _row_spec(H)],
    out_shape=[
        jax.ShapeDtypeStruct((N_PAD, D), jnp.float32),
        jax.ShapeDtypeStruct((N_PAD, H), jnp.float32),
        jax.ShapeDtypeStruct((N_PAD, H), jnp.float32),
        jax.ShapeDtypeStruct((N_PAD, H), jnp.float32),
        jax.ShapeDtypeStruct((N_PAD, H), jnp.float32),
    ],
)


def _tc_final_body(h_ref, a0_ref, a1_ref, wu_ref, bu_ref, wo_ref, bo_ref,
                   out_ref):
    h = h_ref[...]
    u = (_dot(h, wu_ref[0:D, :])
         + _dot(a0_ref[...], wu_ref[D:D + H, :])
         + _dot(a1_ref[...], wu_ref[D + H:2 * D, :])
         + bu_ref[...])
    hn = h + jnp.maximum(u, 0.0)
    out_ref[...] = _dot(hn, wo_ref[...]) + bo_ref[...]


_tc_final = pl.pallas_call(
    _tc_final_body,
    grid=(N_BLKS,),
    in_specs=[
        _row_spec(D), _row_spec(H), _row_spec(H),
        _full_spec(2 * D, D), _full_spec(1, D),
        _full_spec(D, 1), _full_spec(1, 1),
    ],
    out_specs=[_row_spec(1)],
    out_shape=[jax.ShapeDtypeStruct((N_PAD, 1), jnp.float32)],
)


def _sc_body(p0_hbm, p1_hbm, q0_hbm, q1_hbm, dst_hbm, src_hbm, ea_hbm, w3_hbm,
             aggr_hbm, aggr_sh, dstv, srcv, eav, pdv, qsv, w3v, sem):
    c = lax.axis_index("c")
    s = lax.axis_index("s")
    row0 = s * RPT

    pltpu.sync_copy(w3_hbm.at[c], w3v)

    def _zrow(j, _):
        qsv[j, 0:16] = jnp.zeros((16,), jnp.float32)
        qsv[j, 16:32] = jnp.zeros((16,), jnp.float32)
        return 0
    lax.fori_loop(0, CHUNK, _zrow, 0)
    for i in range(RPT // CHUNK):
        pltpu.sync_copy(qsv, aggr_sh.at[pl.ds(row0 + i * CHUNK, CHUNK)])
    rem = RPT - (RPT // CHUNK) * CHUNK
    if rem:
        pltpu.sync_copy(qsv.at[pl.ds(0, rem)],
                        aggr_sh.at[pl.ds(row0 + RPT - rem, rem)])
    plsc.subcore_barrier()

    def _run(p_hbm, q_hbm):
        def _rb(v):
            return v.astype(jnp.bfloat16).astype(jnp.float32)
        w3c = [[_rb(w3v[k, 0:16]), _rb(w3v[k, 16:32])] for k in range(ED)]

        def _chunk(ci, _):
            r0 = s * (EPT // IDX_W) + ci * SUB
            pltpu.sync_copy(dst_hbm.at[pl.ds(r0, SUB)], dstv)
            pltpu.sync_copy(src_hbm.at[pl.ds(r0, SUB)], srcv)
            e0 = s * EPT + ci * CHUNK
            pltpu.sync_copy(ea_hbm.at[pl.ds(e0 * ED, CHUNK * ED)], eav)
            cps = []
            for jj in range(SUB):
                cps.append(pltpu.async_copy(
                    p_hbm.at[dstv.at[jj]], pdv.at[pl.ds(jj * IDX_W, IDX_W)], sem))
                cps.append(pltpu.async_copy(
                    q_hbm.at[srcv.at[jj]], qsv.at[pl.ds(jj * IDX_W, IDX_W)], sem))
            for cp in cps:
                cp.wait()

            def _edge(j4, _):
                ea16 = _rb(eav[pl.ds(j4 * 16, 16)])
                for u in range(4):
                    j = j4 * 4 + u
                    a0 = pdv[j, 0:16] + qsv[j, 0:16]
                    a1 = pdv[j, 16:32] + qsv[j, 16:32]
                    for k in range(ED):
                        ek = ea16[u * ED + k]
                        a0 = a0 + ek * w3c[k][0]
                        a1 = a1 + ek * w3c[k][1]
                    pdv[j, 0:16] = jnp.maximum(a0, 0.0)
                    pdv[j, 16:32] = jnp.maximum(a1, 0.0)
                return 0
            lax.fori_loop(0, CHUNK // 4, _edge, 0)

            for jj in range(SUB):
                pltpu.sync_copy(pdv.at[pl.ds(jj * IDX_W, IDX_W)],
                                aggr_sh.at[dstv.at[jj]], add=True)
            return 0
        lax.fori_loop(0, NCHUNK, _chunk, 0)

    @pl.when(c == 0)
    def _():
        _run(p0_hbm, q0_hbm)

    @pl.when(c == 1)
    def _():
        _run(p1_hbm, q1_hbm)

    plsc.subcore_barrier()
    pltpu.sync_copy(aggr_sh.at[pl.ds(row0, RPT)], aggr_hbm.at[c, pl.ds(row0, RPT)])


@functools.cache
def _get_sc_layer():
    mesh = plsc.VectorSubcoreMesh(core_axis_name="c", subcore_axis_name="s",
                                  num_cores=2, num_subcores=NS)
    return pl.kernel(
        _sc_body,
        out_type=jax.ShapeDtypeStruct((2, N_PAD, H), jnp.float32),
        mesh=mesh,
        compiler_params=pltpu.CompilerParams(use_tc_tiling_on_sc=False),
        scratch_types=[
            pltpu.VMEM_SHARED((N_PAD, H), jnp.float32),
            pltpu.VMEM((SUB, IDX_W), jnp.int32),
            pltpu.VMEM((SUB, IDX_W), jnp.int32),
            pltpu.VMEM((CHUNK * ED,), jnp.float32),
            pltpu.VMEM((CHUNK, H), jnp.float32),
            pltpu.VMEM((CHUNK, H), jnp.float32),
            pltpu.VMEM((ED, H), jnp.float32),
            pltpu.SemaphoreType.DMA,
        ],
    )


def kernel(x, edge_index, edge_attr, params):
    f32 = jnp.float32
    win = params["W_in"].astype(f32)
    bin_ = params["b_in"].astype(f32).reshape(1, D)
    wm = params["W_msg"].astype(f32)
    bm = params["b_msg"].astype(f32)
    wu = params["W_upd"].astype(f32)
    bu = params["b_upd"].astype(f32)
    wo = params["W_pred"].astype(f32)
    bo = params["b_pred"].astype(f32).reshape(1, 1)

    x_pad = jnp.zeros((N_PAD, x.shape[1]), f32).at[:N_NODES].set(x)
    pad_idx = jnp.full((E_PAD - E_EDGES,), N_NODES, jnp.int32)
    dst_p = jnp.concatenate([edge_index[1], pad_idx]).reshape(E_PAD // IDX_W, IDX_W)
    src_p = jnp.concatenate([edge_index[0], pad_idx]).reshape(E_PAD // IDX_W, IDX_W)
    ea_p = jnp.zeros((E_PAD, ED), f32).at[:E_EDGES].set(edge_attr).reshape(E_PAD * ED)

    h, p0, p1, q0, q1 = _tc_init(
        x_pad, win, bin_, wm[0, :D], bm[0].reshape(1, D), wm[0, D:2 * D])
    out = None
    for l in range(N_LAYERS):
        w3s = jnp.stack([wm[l, 2 * D:, :H], wm[l, 2 * D:, H:]])
        aggr = _get_sc_layer()(p0, p1, q0, q1, dst_p, src_p, ea_p, w3s)
        if l < N_LAYERS - 1:
            h, p0, p1, q0, q1 = _tc_upd(
                h, aggr[0], aggr[1], wu[l], bu[l].reshape(1, D),
                wm[l + 1, :D], bm[l + 1].reshape(1, D), wm[l + 1, D:2 * D])
        else:
            (out,) = _tc_final(
                h, aggr[0], aggr[1], wu[l], bu[l].reshape(1, D), wo, bo)
    return out[:N_NODES]


# speedup vs baseline: 3.9900x; 1.1677x over previous
import functools

import jax
import jax.numpy as jnp
from jax import lax
from jax.experimental import pallas as pl
from jax.experimental.pallas import tpu as pltpu
from jax.experimental.pallas import tpu_sc as plsc

N_NODES = 50000
E_EDGES = 800000
D = 64
H = 32
ED = 4
N_LAYERS = 4

ROW_BLK = 512
N_PAD = 50176
N_BLKS = N_PAD // ROW_BLK

IDX_W = 128
CHUNK = 256
SUB = CHUNK // IDX_W
NS = 16
NCHUNK_ALL = E_EDGES // CHUNK
NCHUNK_BASE = NCHUNK_ALL // NS
NCHUNK_EXTRA = NCHUNK_ALL - NCHUNK_BASE * NS
RPT = N_PAD // NS


def _dot(a, b):
    return jnp.dot(a.astype(jnp.bfloat16), b.astype(jnp.bfloat16),
                   preferred_element_type=jnp.float32)


def _tc_init_body(x_ref, win_ref, bin_ref, wp_ref, bp_ref, wq_ref,
                  h_ref, p0_ref, p1_ref, q0_ref, q1_ref):
    h = _dot(x_ref[...], win_ref[...]) + bin_ref[...]
    h_ref[...] = h
    p = _dot(h, wp_ref[...]) + bp_ref[...]
    q = _dot(h, wq_ref[...])
    p0_ref[...] = p[:, :H]
    p1_ref[...] = p[:, H:]
    q0_ref[...] = q[:, :H]
    q1_ref[...] = q[:, H:]


def _row_spec(w):
    return pl.BlockSpec((ROW_BLK, w), lambda i: (i, 0))


def _full_spec(r, w):
    return pl.BlockSpec((r, w), lambda i: (0, 0))


_tc_init = pl.pallas_call(
    _tc_init_body,
    grid=(N_BLKS,),
    in_specs=[
        _row_spec(11),
        _full_spec(11, D), _full_spec(1, D),
        _full_spec(D, D), _full_spec(1, D), _full_spec(D, D),
    ],
    out_specs=[_row_spec(D), _row_spec(H), _row_spec(H), _row_spec(H), _row_spec(H)],
    out_shape=[
        jax.ShapeDtypeStruct((N_PAD, D), jnp.float32),
        jax.ShapeDtypeStruct((N_PAD, H), jnp.float32),
        jax.ShapeDtypeStruct((N_PAD, H), jnp.float32),
        jax.ShapeDtypeStruct((N_PAD, H), jnp.float32),
        jax.ShapeDtypeStruct((N_PAD, H), jnp.float32),
    ],
)


def _tc_upd_body(h_ref, a0_ref, a1_ref, wu_ref, bu_ref, wp_ref, bp_ref, wq_ref,
                 hn_ref, p0_ref, p1_ref, q0_ref, q1_ref):
    h = h_ref[...]
    u = (_dot(h, wu_ref[0:D, :])
         + _dot(a0_ref[...], wu_ref[D:D + H, :])
         + _dot(a1_ref[...], wu_ref[D + H:2 * D, :])
         + bu_ref[...])
    hn = h + jnp.maximum(u, 0.0)
    hn_ref[...] = hn
    p = _dot(hn, wp_ref[...]) + bp_ref[...]
    q = _dot(hn, wq_ref[...])
    p0_ref[...] = p[:, :H]
    p1_ref[...] = p[:, H:]
    q0_ref[...] = q[:, :H]
    q1_ref[...] = q[:, H:]


_tc_upd = pl.pallas_call(
    _tc_upd_body,
    grid=(N_BLKS,),
    in_specs=[
        _row_spec(D), _row_spec(H), _row_spec(H),
        _full_spec(2 * D, D), _full_spec(1, D),
        _full_spec(D, D), _full_spec(1, D), _full_spec(D, D),
    ],
    out_specs=[_row_spec(D), _row_spec(H), _row_spec(H), _row_spec(H), _row_spec(H)],
    out_shape=[
        jax.ShapeDtypeStruct((N_PAD, D), jnp.float32),
        jax.ShapeDtypeStruct((N_PAD, H), jnp.float32),
        jax.ShapeDtypeStruct((N_PAD, H), jnp.float32),
        jax.ShapeDtypeStruct((N_PAD, H), jnp.float32),
        jax.ShapeDtypeStruct((N_PAD, H), jnp.float32),
    ],
)


def _tc_final_body(h_ref, a0_ref, a1_ref, wu_ref, bu_ref, wo_ref, bo_ref,
                   out_ref):
    h = h_ref[...]
    u = (_dot(h, wu_ref[0:D, :])
         + _dot(a0_ref[...], wu_ref[D:D + H, :])
         + _dot(a1_ref[...], wu_ref[D + H:2 * D, :])
         + bu_ref[...])
    hn = h + jnp.maximum(u, 0.0)
    out_ref[...] = _dot(hn, wo_ref[...]) + bo_ref[...]


_tc_final = pl.pallas_call(
    _tc_final_body,
    grid=(N_BLKS,),
    in_specs=[
        _row_spec(D), _row_spec(H), _row_spec(H),
        _full_spec(2 * D, D), _full_spec(1, D),
        _full_spec(D, 1), _full_spec(1, 1),
    ],
    out_specs=[_row_spec(1)],
    out_shape=[jax.ShapeDtypeStruct((N_PAD, 1), jnp.float32)],
)


def _sc_body(p0_hbm, p1_hbm, q0_hbm, q1_hbm, ei_hbm, ea_hbm, w3_hbm,
             aggr_hbm, aggr_sh, dstv, srcv, eav, pdv, qsv, w3v, sem):
    c = lax.axis_index("c")
    s = lax.axis_index("s")
    row0 = s * RPT

    pltpu.sync_copy(w3_hbm.at[c], w3v)

    def _zrow(j, _):
        qsv[j, 0:16] = jnp.zeros((16,), jnp.float32)
        qsv[j, 16:32] = jnp.zeros((16,), jnp.float32)
        return 0
    lax.fori_loop(0, CHUNK, _zrow, 0)
    for i in range(RPT // CHUNK):
        pltpu.sync_copy(qsv, aggr_sh.at[pl.ds(row0 + i * CHUNK, CHUNK)])
    rem = RPT - (RPT // CHUNK) * CHUNK
    if rem:
        pltpu.sync_copy(qsv.at[pl.ds(0, rem)],
                        aggr_sh.at[pl.ds(row0 + RPT - rem, rem)])
    plsc.subcore_barrier()

    def _run(p_hbm, q_hbm):
        def _rb(v):
            return v.astype(jnp.bfloat16).astype(jnp.float32)
        w3c = [[_rb(w3v[k, 0:16]), _rb(w3v[k, 16:32])] for k in range(ED)]

        c0 = s * NCHUNK_BASE + jnp.minimum(s, NCHUNK_EXTRA)
        nch = jnp.where(s < NCHUNK_EXTRA, NCHUNK_BASE + 1, NCHUNK_BASE)

        def _chunk(ci, _):
            g = c0 + ci
            r0 = g * SUB
            pltpu.sync_copy(ei_hbm.at[1, pl.ds(r0, SUB)], dstv)
            pltpu.sync_copy(ei_hbm.at[0, pl.ds(r0, SUB)], srcv)
            pltpu.sync_copy(ea_hbm.at[pl.ds(g * (CHUNK * ED), CHUNK * ED)], eav)
            cps = []
            for jj in range(SUB):
                cps.append(pltpu.async_copy(
                    p_hbm.at[dstv.at[jj]], pdv.at[pl.ds(jj * IDX_W, IDX_W)], sem))
                cps.append(pltpu.async_copy(
                    q_hbm.at[srcv.at[jj]], qsv.at[pl.ds(jj * IDX_W, IDX_W)], sem))
            for cp in cps:
                cp.wait()

            def _edge(j4, _):
                ea16 = _rb(eav[pl.ds(j4 * 16, 16)])
                for u in range(4):
                    j = j4 * 4 + u
                    a0 = pdv[j, 0:16] + qsv[j, 0:16]
                    a1 = pdv[j, 16:32] + qsv[j, 16:32]
                    for k in range(ED):
                        ek = ea16[u * ED + k]
                        a0 = a0 + ek * w3c[k][0]
                        a1 = a1 + ek * w3c[k][1]
                    pdv[j, 0:16] = jnp.maximum(a0, 0.0)
                    pdv[j, 16:32] = jnp.maximum(a1, 0.0)
                return 0
            lax.fori_loop(0, CHUNK // 4, _edge, 0)

            for jj in range(SUB):
                pltpu.sync_copy(pdv.at[pl.ds(jj * IDX_W, IDX_W)],
                                aggr_sh.at[dstv.at[jj]], add=True)
            return 0
        lax.fori_loop(0, nch, _chunk, 0)

    @pl.when(c == 0)
    def _():
        _run(p0_hbm, q0_hbm)

    @pl.when(c == 1)
    def _():
        _run(p1_hbm, q1_hbm)

    plsc.subcore_barrier()
    pltpu.sync_copy(aggr_sh.at[pl.ds(row0, RPT)], aggr_hbm.at[c, pl.ds(row0, RPT)])


@functools.cache
def _get_sc_layer():
    mesh = plsc.VectorSubcoreMesh(core_axis_name="c", subcore_axis_name="s",
                                  num_cores=2, num_subcores=NS)
    return pl.kernel(
        _sc_body,
        out_type=jax.ShapeDtypeStruct((2, N_PAD, H), jnp.float32),
        mesh=mesh,
        compiler_params=pltpu.CompilerParams(use_tc_tiling_on_sc=False),
        scratch_types=[
            pltpu.VMEM_SHARED((N_PAD, H), jnp.float32),
            pltpu.VMEM((SUB, IDX_W), jnp.int32),
            pltpu.VMEM((SUB, IDX_W), jnp.int32),
            pltpu.VMEM((CHUNK * ED,), jnp.float32),
            pltpu.VMEM((CHUNK, H), jnp.float32),
            pltpu.VMEM((CHUNK, H), jnp.float32),
            pltpu.VMEM((ED, H), jnp.float32),
            pltpu.SemaphoreType.DMA,
        ],
    )


def kernel(x, edge_index, edge_attr, params):
    f32 = jnp.float32
    win = params["W_in"].astype(f32)
    bin_ = params["b_in"].astype(f32).reshape(1, D)
    wm = params["W_msg"].astype(f32)
    bm = params["b_msg"].astype(f32)
    wu = params["W_upd"].astype(f32)
    bu = params["b_upd"].astype(f32)
    wo = params["W_pred"].astype(f32)
    bo = params["b_pred"].astype(f32).reshape(1, 1)

    x_pad = jnp.zeros((N_PAD, x.shape[1]), f32).at[:N_NODES].set(x)
    ei_r = edge_index.reshape(2, E_EDGES // IDX_W, IDX_W)
    ea_r = edge_attr.astype(f32).reshape(E_EDGES * ED)

    h, p0, p1, q0, q1 = _tc_init(
        x_pad, win, bin_, wm[0, :D], bm[0].reshape(1, D), wm[0, D:2 * D])
    out = None
    for l in range(N_LAYERS):
        w3s = jnp.stack([wm[l, 2 * D:, :H], wm[l, 2 * D:, H:]])
        aggr = _get_sc_layer()(p0, p1, q0, q1, ei_r, ea_r, w3s)
        if l < N_LAYERS - 1:
            h, p0, p1, q0, q1 = _tc_upd(
                h, aggr[0], aggr[1], wu[l], bu[l].reshape(1, D),
                wm[l + 1, :D], bm[l + 1].reshape(1, D), wm[l + 1, D:2 * D])
        else:
            (out,) = _tc_final(
                h, aggr[0], aggr[1], wu[l], bu[l].reshape(1, D), wo, bo)
    return out[:N_NODES]
